# baseline (speedup 1.0000x reference)
import functools
import math

import jax
import jax.numpy as jnp
from jax import lax
from jax.experimental import pallas as pl
from jax.experimental.pallas import tpu as pltpu
from jax.experimental.pallas import tpu_sc as plsc


def _make_sc_kernel(BT, V, E, NW, CH, NB):
    N = BT // NW
    NCH = N // CH
    T = NCH // NB
    assert NCH % NB == 0
    scale = float(math.sqrt(E))
    mesh = plsc.VectorSubcoreMesh(core_axis_name="c", subcore_axis_name="s")
    NC = 2

    @functools.partial(
        pl.kernel,
        mesh=mesh,
        out_type=jax.ShapeDtypeStruct((BT, E), jnp.float32),
        scratch_types=[
            pltpu.VMEM((NCH, CH), jnp.int32),
            pltpu.VMEM((NB, CH, E), jnp.float32),
            pltpu.VMEM((NB, CH, E), jnp.float32),
            pltpu.SemaphoreType.DMA((NB,)),
            pltpu.SemaphoreType.DMA((NB,)),
        ],
        compiler_params=pltpu.CompilerParams(use_tc_tiling_on_sc=False),
    )
    def k(x_hbm, table_hbm, out_hbm, idx_v, gbuf, sbuf, gsem, ssem):
        wid = lax.axis_index("s") * NC + lax.axis_index("c")
        pltpu.sync_copy(x_hbm.at[wid], idx_v)
        base = wid * N

        def gather_start(j, b):
            for kk in range(CH // 16):
                iv = idx_v[j, pl.ds(kk * 16, 16)]
                pltpu.async_copy(
                    table_hbm.at[iv],
                    gbuf.at[b, pl.ds(kk * 16, 16)],
                    gsem.at[b],
                )

        def gather_wait(b):
            pltpu.make_async_copy(
                table_hbm.at[pl.ds(0, CH)], gbuf.at[b], gsem.at[b]).wait()

        def store_start(j, b):
            pltpu.async_copy(
                sbuf.at[b], out_hbm.at[pl.ds(base + j * CH, CH)], ssem.at[b])

        def store_wait(b):
            pltpu.make_async_copy(
                sbuf.at[b], out_hbm.at[pl.ds(base, CH)], ssem.at[b]).wait()

        for b in range(NB):
            gather_start(b, b)

        def outer(t, carry):
            for b in range(NB):
                j = t * NB + b

                @pl.when(t > 0)
                def _w():
                    store_wait(b)

                gather_wait(b)

                def scale_row(r, c2, b=b):
                    for cc in range(E // 16):
                        sl = pl.ds(cc * 16, 16)
                        sbuf[b, r, sl] = gbuf[b, r, sl] * scale
                    return c2

                lax.fori_loop(0, CH, scale_row, 0, unroll=8)

                @pl.when(t < T - 1)
                def _g():
                    gather_start(j + NB, b)

                store_start(j, b)
            return carry

        lax.fori_loop(0, T, outer, 0)
        for b in range(NB):
            store_wait(b)

    return k


def kernel(x, table):
    B, L = x.shape
    V, E = table.shape
    BT = B * L
    NW = 32
    CH = 128
    NB = 4
    x_r = x.reshape(NW, BT // (NW * CH), CH)
    k = _make_sc_kernel(BT, V, E, NW, CH, NB)
    out = k(x_r, table)
    return out.reshape(B, L, E)


# speedup vs baseline: 1.0373x; 1.0373x over previous
import functools
import math

import jax
import jax.numpy as jnp
from jax import lax
from jax.experimental import pallas as pl
from jax.experimental.pallas import tpu as pltpu
from jax.experimental.pallas import tpu_sc as plsc


def _make_sc_kernel(BT, V, E, NW, CH, NB):
    N = BT // NW
    NCH = N // CH
    T = NCH // NB
    assert NCH % NB == 0
    scale = float(math.sqrt(E))
    mesh = plsc.VectorSubcoreMesh(core_axis_name="c", subcore_axis_name="s")
    NC = 2

    @functools.partial(
        pl.kernel,
        mesh=mesh,
        out_type=jax.ShapeDtypeStruct((BT, E), jnp.float32),
        scratch_types=[
            pltpu.VMEM((NCH, CH), jnp.int32),
            pltpu.VMEM((NB, CH, E), jnp.float32),
            pltpu.VMEM((NB, CH, E), jnp.float32),
            pltpu.SemaphoreType.DMA((NB,)),
            pltpu.SemaphoreType.DMA((NB,)),
        ],
        compiler_params=pltpu.CompilerParams(use_tc_tiling_on_sc=False),
    )
    def k(x_hbm, table_hbm, out_hbm, idx_v, gbuf, sbuf, gsem, ssem):
        wid = lax.axis_index("s") * NC + lax.axis_index("c")
        pltpu.sync_copy(x_hbm.at[wid], idx_v)
        base = wid * N

        def gather_start(j, b):
            pltpu.async_copy(table_hbm.at[idx_v.at[j]], gbuf.at[b], gsem.at[b])

        def gather_wait(b):
            pltpu.make_async_copy(
                table_hbm.at[pl.ds(0, CH)], gbuf.at[b], gsem.at[b]).wait()

        def store_start(j, b):
            pltpu.async_copy(
                sbuf.at[b], out_hbm.at[pl.ds(base + j * CH, CH)], ssem.at[b])

        def store_wait(b):
            pltpu.make_async_copy(
                sbuf.at[b], out_hbm.at[pl.ds(0, CH)], ssem.at[b]).wait()

        for b in range(NB):
            gather_start(b, b)

        def outer(t, carry):
            for b in range(NB):
                j = t * NB + b

                @pl.when(t > 0)
                def _w():
                    store_wait(b)

                gather_wait(b)

                def scale_row(r, c2, b=b):
                    for cc in range(E // 16):
                        sl = pl.ds(cc * 16, 16)
                        sbuf[b, r, sl] = gbuf[b, r, sl] * scale
                    return c2

                lax.fori_loop(0, CH, scale_row, 0, unroll=8)

                @pl.when(t < T - 1)
                def _g():
                    gather_start(j + NB, b)

                store_start(j, b)
            return carry

        lax.fori_loop(0, T, outer, 0)
        for b in range(NB):
            store_wait(b)

    return k


def kernel(x, table):
    B, L = x.shape
    V, E = table.shape
    BT = B * L
    NW = 32
    CH = 128
    NB = 5
    x_r = x.reshape(NW, BT // (NW * CH), CH)
    k = _make_sc_kernel(BT, V, E, NW, CH, NB)
    out = k(x_r, table)
    return out.reshape(B, L, E)


# speedup vs baseline: 1.1059x; 1.0662x over previous
import functools
import math

import jax
import jax.numpy as jnp
from jax import lax
from jax.experimental import pallas as pl
from jax.experimental.pallas import tpu as pltpu
from jax.experimental.pallas import tpu_sc as plsc


def _make_sc_kernel(BT, V, E, NW, CH, NB):
    N = BT // NW
    NCH = N // CH
    T = NCH // NB
    assert NCH % NB == 0
    scale = float(math.sqrt(E))
    mesh = plsc.VectorSubcoreMesh(core_axis_name="c", subcore_axis_name="s")
    NC = 2

    @functools.partial(
        pl.kernel,
        mesh=mesh,
        out_type=jax.ShapeDtypeStruct((BT, E), jnp.float32),
        scratch_types=[
            pltpu.VMEM((NCH, CH), jnp.int32),
            pltpu.VMEM((NB, CH, E), jnp.float32),
            pltpu.VMEM((NB, CH, E), jnp.float32),
            pltpu.SemaphoreType.DMA((NB,)),
            pltpu.SemaphoreType.DMA((NB,)),
        ],
        compiler_params=pltpu.CompilerParams(use_tc_tiling_on_sc=False),
    )
    def k(x_hbm, table_hbm, out_hbm, idx_v, gbuf, sbuf, gsem, ssem):
        wid = lax.axis_index("s") * NC + lax.axis_index("c")
        pltpu.sync_copy(x_hbm.at[wid], idx_v)
        base = wid * N

        def gather_start(j, b):
            pltpu.async_copy(table_hbm.at[idx_v.at[j]], gbuf.at[b], gsem.at[b])

        def gather_wait(b):
            pltpu.make_async_copy(
                table_hbm.at[pl.ds(0, CH)], gbuf.at[b], gsem.at[b]).wait()

        def store_start(j, b):
            pltpu.async_copy(
                sbuf.at[b], out_hbm.at[pl.ds(base + j * CH, CH)], ssem.at[b])

        def store_wait(b):
            pltpu.make_async_copy(
                sbuf.at[b], out_hbm.at[pl.ds(0, CH)], ssem.at[b]).wait()

        for b in range(NB):
            gather_start(b, b)

        def outer(t, carry):
            for b in range(NB):
                j = t * NB + b

                @pl.when(t > 0)
                def _w():
                    store_wait(b)

                gather_wait(b)

                @plsc.parallel_loop(0, CH, unroll=8)
                def scale_row(r, b=b):
                    for cc in range(E // 16):
                        sl = pl.ds(cc * 16, 16)
                        sbuf[b, r, sl] = gbuf[b, r, sl] * scale

                @pl.when(t < T - 1)
                def _g():
                    gather_start(j + NB, b)

                store_start(j, b)
            return carry

        lax.fori_loop(0, T, outer, 0)
        for b in range(NB):
            store_wait(b)

    return k


def kernel(x, table):
    B, L = x.shape
    V, E = table.shape
    BT = B * L
    NW = 32
    CH = 128
    NB = 5
    x_r = x.reshape(NW, BT // (NW * CH), CH)
    k = _make_sc_kernel(BT, V, E, NW, CH, NB)
    out = k(x_r, table)
    return out.reshape(B, L, E)
